# baseline (speedup 1.0000x reference)
import functools

import jax
import jax.numpy as jnp
from jax import lax
from jax.experimental import pallas as pl
from jax.experimental.pallas import tpu as pltpu
from jax.experimental.pallas import tpu_sc as plsc

NC = 2
NS = 16
NW = NC * NS
L = 16


def _proj_body(emb_ref, w1_ref, b1_ref, a_ref, b_ref):
    h = emb_ref.shape[1]
    emb = emb_ref[...]
    w1 = w1_ref[...]
    dn = (((1,), (1,)), ((), ()))
    a_ref[...] = lax.dot_general(
        emb, w1[:, :h], dn, preferred_element_type=jnp.float32) + b1_ref[...]
    b_ref[...] = lax.dot_general(
        emb, w1[:, h:], dn, preferred_element_type=jnp.float32)


def _init_body(o_ref):
    o_ref[...] = jnp.zeros_like(o_ref)


def kernel(node_emb, edge_index, batch, W1, b1, W2, b2):
    n, h = node_emb.shape
    e = edge_index.shape[1]
    del batch

    blk_p = 2000
    a_mat, b_mat = pl.pallas_call(
        _proj_body,
        grid=(n // blk_p,),
        in_specs=[
            pl.BlockSpec((blk_p, h), lambda i: (i, 0)),
            pl.BlockSpec((h, 2 * h), lambda i: (0, 0)),
            pl.BlockSpec((1, h), lambda i: (0, 0)),
        ],
        out_specs=[
            pl.BlockSpec((blk_p, h), lambda i: (i, 0)),
            pl.BlockSpec((blk_p, h), lambda i: (i, 0)),
        ],
        out_shape=[
            jax.ShapeDtypeStruct((n, h), jnp.float32),
            jax.ShapeDtypeStruct((n, h), jnp.float32),
        ],
    )(node_emb, W1, b1.reshape(1, h))

    G = 128
    e_pad = ((e + NW * G - 1) // (NW * G)) * (NW * G)
    per_w = e_pad // NW
    nchunk = per_w // G
    chunks_tot = e_pad // G
    pad = e_pad - e
    dpad = jnp.arange(pad, dtype=jnp.int32) % n
    row_p = jnp.concatenate([edge_index[0], dpad])
    col_p = jnp.concatenate([edge_index[1], dpad])
    w2_vec = W2.reshape(h)
    b2_vec = jnp.broadcast_to(b2.reshape(1), (L,)).astype(jnp.float32)

    EN0, EN1 = nchunk, nchunk
    dchunks = 80
    schunks_tot = chunks_tot + dchunks
    spw = schunks_tot // 16
    SN0 = spw // 2
    SN1 = spw - SN0

    mesh = plsc.VectorSubcoreMesh(core_axis_name="c", subcore_axis_name="s")

    @functools.partial(
        pl.kernel,
        mesh=mesh,
        compiler_params=pltpu.CompilerParams(needs_layout_passes=False),
        out_type=(
            jax.ShapeDtypeStruct((chunks_tot, G), jnp.float32),
            jax.ShapeDtypeStruct((chunks_tot, G), jnp.int32),
        ),
        scratch_types=[
            pltpu.VMEM((2, G), jnp.int32),
            pltpu.VMEM((2, G), jnp.int32),
            pltpu.VMEM((2, G, h), jnp.float32),
            pltpu.VMEM((2, G, h), jnp.float32),
            pltpu.VMEM((2, G), jnp.float32),
            pltpu.VMEM((2, G), jnp.int32),
            pltpu.VMEM((h,), jnp.float32),
            pltpu.VMEM((L,), jnp.float32),
            pltpu.VMEM((L, L), jnp.float32),
            pltpu.SemaphoreType.DMA,
            pltpu.SemaphoreType.DMA,
            pltpu.SemaphoreType.DMA,
            pltpu.SemaphoreType.DMA,
        ],
    )
    def edge_kernel(a_hbm, b_hbm, row_hbm, col_hbm, w2_hbm, b2_hbm,
                    prob_hbm, fidx_hbm,
                    idxa_v, idxb_v, arow_v, brow_v, prob_v, fidx_v,
                    w2_v, b2_v, m_v, sem_a, sem_b, sem_sp, sem_sf):
        cid = lax.axis_index("c")
        sid = lax.axis_index("s")
        cbase = jnp.where(cid == 0, sid * EN0, 16 * EN0 + sid * EN1)
        ccount = jnp.where(cid == 0, EN0, EN1)
        pltpu.sync_copy(w2_hbm, w2_v)
        pltpu.sync_copy(b2_hbm, b2_v)
        b2v = b2_v[...]
        rowi = lax.iota(jnp.int32, L)

        def prefetch(k, slot):
            off = (cbase + k) * G
            pltpu.sync_copy(row_hbm.at[pl.ds(off, G)], idxa_v.at[slot])
            pltpu.sync_copy(col_hbm.at[pl.ds(off, G)], idxb_v.at[slot])
            pltpu.async_copy(a_hbm.at[idxa_v.at[slot]], arow_v.at[slot],
                             sem_a)
            pltpu.async_copy(b_hbm.at[idxb_v.at[slot]], brow_v.at[slot],
                             sem_b)

        def wait_gather(slot):
            pltpu.make_async_copy(a_hbm.at[idxa_v.at[slot]],
                                  arow_v.at[slot], sem_a).wait()
            pltpu.make_async_copy(b_hbm.at[idxb_v.at[slot]],
                                  brow_v.at[slot], sem_b).wait()

        def wait_store(slot):
            pltpu.make_async_copy(prob_v.at[slot], prob_hbm.at[0],
                                  sem_sp).wait()
            pltpu.make_async_copy(fidx_v.at[slot], fidx_hbm.at[0],
                                  sem_sf).wait()

        prefetch(0, 0)

        def chunk_body(k, carry):
            p = lax.rem(k, 2)
            q = 1 - p

            @pl.when(k + 1 < ccount)
            def _():
                prefetch(k + 1, q)

            wait_gather(p)

            @pl.when(k >= 2)
            def _():
                wait_store(p)

            def group_body(g, carry2):
                e0 = g * L
                for ee in range(L):
                    acc = None
                    for j in range(h // L):
                        va = arow_v[p, e0 + ee, pl.ds(j * L, L)]
                        vb = brow_v[p, e0 + ee, pl.ds(j * L, L)]
                        x = va + vb
                        el = jnp.where(x > 0, x, jnp.exp(x) - 1.0)
                        t = el * w2_v[pl.ds(j * L, L)]
                        acc = t if acc is None else acc + t
                    m_v[ee] = acc
                s = None
                for j in range(L):
                    v = plsc.load_gather(
                        m_v, [rowi, jnp.full((L,), j, jnp.int32)])
                    s = v if s is None else s + v
                logit = s + b2v
                pv = 1.0 / (1.0 + jnp.exp(-logit))
                ra = idxa_v[p, pl.ds(e0, L)]
                rb = idxb_v[p, pl.ds(e0, L)]
                pv = jnp.where(ra == rb, jnp.float32(1.0), pv)
                prob_v[p, pl.ds(e0, L)] = pv
                fidx_v[p, pl.ds(e0, L)] = ra * n + rb
                return carry2

            lax.fori_loop(0, G // L, group_body, 0)
            crow = cbase + k
            pltpu.async_copy(prob_v.at[p], prob_hbm.at[crow], sem_sp)
            pltpu.async_copy(fidx_v.at[p], fidx_hbm.at[crow], sem_sf)
            return carry

        lax.fori_loop(0, ccount, chunk_body, 0)
        wait_store(0)
        wait_store(1)

    prob, fidx = edge_kernel(a_mat, b_mat, row_p, col_p, w2_vec, b2_vec)

    blk_i = 4_194_304
    adj0 = pl.pallas_call(
        _init_body,
        grid=(pl.cdiv(n * n, blk_i),),
        out_specs=pl.BlockSpec((blk_i,), lambda i: (i,)),
        out_shape=jax.ShapeDtypeStruct((n * n,), jnp.float32),
    )()

    diag_fidx = ((jnp.arange(dchunks * G, dtype=jnp.int32) % n)
                 * (n + 1)).reshape(dchunks, G)
    diag_prob = jnp.ones((dchunks, G), jnp.float32)
    sprob = jnp.concatenate([prob, diag_prob], axis=0)
    sfidx = jnp.concatenate([fidx, diag_fidx], axis=0)

    @functools.partial(
        pl.kernel,
        mesh=mesh,
        out_type=(),
        scratch_types=[
            pltpu.VMEM((spw, G), jnp.float32),
            pltpu.VMEM((spw, G), jnp.int32),
            pltpu.SemaphoreType.DMA,
            pltpu.SemaphoreType.DMA,
        ],
    )
    def scatter_kernel(prob_hbm, fidx_hbm, adj_hbm, pv, iv, sem_l, sem_s):
        cid = lax.axis_index("c")
        sid = lax.axis_index("s")
        cbase = jnp.where(cid == 0, sid * SN0, 16 * SN0 + sid * SN1)
        ccount = jnp.where(cid == 0, SN0, SN1)

        def issue_load(k, carry):
            pltpu.async_copy(prob_hbm.at[cbase + k], pv.at[k], sem_l)
            pltpu.async_copy(fidx_hbm.at[cbase + k], iv.at[k], sem_l)
            return carry

        def drain_load(k, carry):
            pltpu.make_async_copy(prob_hbm.at[0], pv.at[0], sem_l).wait()
            pltpu.make_async_copy(fidx_hbm.at[0], iv.at[0], sem_l).wait()
            return carry

        def issue_scatter(k, carry):
            pltpu.async_copy(pv.at[k], adj_hbm.at[iv.at[k]], sem_s)
            return carry

        def drain_scatter(k, carry):
            pltpu.make_async_copy(pv.at[0], adj_hbm.at[iv.at[0]],
                                  sem_s).wait()
            return carry

        lax.fori_loop(0, ccount, issue_load, 0)
        lax.fori_loop(0, ccount, drain_load, 0)
        lax.fori_loop(0, ccount, issue_scatter, 0)
        lax.fori_loop(0, ccount, drain_scatter, 0)

    adj_ref = jax.new_ref(adj0)
    scatter_kernel(sprob, sfidx, adj_ref)
    return jax.freeze(adj_ref).reshape(n, n)


# speedup vs baseline: 1.0619x; 1.0619x over previous
import functools

import jax
import jax.numpy as jnp
from jax import lax
from jax.experimental import pallas as pl
from jax.experimental.pallas import tpu as pltpu
from jax.experimental.pallas import tpu_sc as plsc

NC = 2
NS = 16
NW = NC * NS
L = 16


def _proj_body(emb_ref, w1_ref, b1_ref, a_ref, b_ref):
    h = emb_ref.shape[1]
    emb = emb_ref[...]
    w1 = w1_ref[...]
    dn = (((1,), (1,)), ((), ()))
    a_ref[...] = lax.dot_general(
        emb, w1[:, :h], dn, preferred_element_type=jnp.float32) + b1_ref[...]
    b_ref[...] = lax.dot_general(
        emb, w1[:, h:], dn, preferred_element_type=jnp.float32)


def _init_body(o_ref):
    o_ref[...] = jnp.zeros_like(o_ref)


def kernel(node_emb, edge_index, batch, W1, b1, W2, b2):
    n, h = node_emb.shape
    e = edge_index.shape[1]
    del batch

    blk_p = 2000
    a_mat, b_mat = pl.pallas_call(
        _proj_body,
        grid=(n // blk_p,),
        in_specs=[
            pl.BlockSpec((blk_p, h), lambda i: (i, 0)),
            pl.BlockSpec((h, 2 * h), lambda i: (0, 0)),
            pl.BlockSpec((1, h), lambda i: (0, 0)),
        ],
        out_specs=[
            pl.BlockSpec((blk_p, h), lambda i: (i, 0)),
            pl.BlockSpec((blk_p, h), lambda i: (i, 0)),
        ],
        out_shape=[
            jax.ShapeDtypeStruct((n, h), jnp.float32),
            jax.ShapeDtypeStruct((n, h), jnp.float32),
        ],
    )(node_emb, W1, b1.reshape(1, h))

    G = 128
    e_pad = ((e + NW * G - 1) // (NW * G)) * (NW * G)
    per_w = e_pad // NW
    nchunk = per_w // G
    chunks_tot = e_pad // G
    pad = e_pad - e
    dpad = jnp.arange(pad, dtype=jnp.int32) % n
    row_p = jnp.concatenate([edge_index[0], dpad])
    col_p = jnp.concatenate([edge_index[1], dpad])
    w2_vec = W2.reshape(h)
    b2_vec = jnp.broadcast_to(b2.reshape(1), (L,)).astype(jnp.float32)

    EN0, EN1 = nchunk, nchunk
    dchunks = 80
    schunks_tot = chunks_tot + dchunks
    spw = schunks_tot // 16
    SN0 = spw // 2
    SN1 = spw - SN0

    mesh = plsc.VectorSubcoreMesh(core_axis_name="c", subcore_axis_name="s")

    @functools.partial(
        pl.kernel,
        mesh=mesh,
        compiler_params=pltpu.CompilerParams(needs_layout_passes=False),
        out_type=(
            jax.ShapeDtypeStruct((chunks_tot, G), jnp.float32),
            jax.ShapeDtypeStruct((chunks_tot, G), jnp.int32),
        ),
        scratch_types=[
            pltpu.VMEM((nchunk * G,), jnp.int32),
            pltpu.VMEM((nchunk * G,), jnp.int32),
            pltpu.VMEM((2, G, h), jnp.float32),
            pltpu.VMEM((2, G, h), jnp.float32),
            pltpu.VMEM((2, G), jnp.float32),
            pltpu.VMEM((2, G), jnp.int32),
            pltpu.VMEM((h,), jnp.float32),
            pltpu.VMEM((L,), jnp.float32),
            pltpu.VMEM((L, L), jnp.float32),
            pltpu.SemaphoreType.DMA,
            pltpu.SemaphoreType.DMA,
            pltpu.SemaphoreType.DMA,
            pltpu.SemaphoreType.DMA,
        ],
    )
    def edge_kernel(a_hbm, b_hbm, row_hbm, col_hbm, w2_hbm, b2_hbm,
                    prob_hbm, fidx_hbm,
                    idxa_v, idxb_v, arow_v, brow_v, prob_v, fidx_v,
                    w2_v, b2_v, m_v, sem_a, sem_b, sem_sp, sem_sf):
        cid = lax.axis_index("c")
        sid = lax.axis_index("s")
        wid = sid * NC + cid
        cbase = wid * nchunk
        pltpu.sync_copy(w2_hbm, w2_v)
        pltpu.sync_copy(b2_hbm, b2_v)
        pltpu.sync_copy(row_hbm.at[pl.ds(cbase * G, nchunk * G)], idxa_v)
        pltpu.sync_copy(col_hbm.at[pl.ds(cbase * G, nchunk * G)], idxb_v)
        b2v = b2_v[...]
        rowi = lax.iota(jnp.int32, L)

        def prefetch(k, slot):
            pltpu.async_copy(a_hbm.at[idxa_v.at[pl.ds(k * G, G)]],
                             arow_v.at[slot], sem_a)
            pltpu.async_copy(b_hbm.at[idxb_v.at[pl.ds(k * G, G)]],
                             brow_v.at[slot], sem_b)

        def wait_gather(slot):
            pltpu.make_async_copy(a_hbm.at[idxa_v.at[pl.ds(0, G)]],
                                  arow_v.at[slot], sem_a).wait()
            pltpu.make_async_copy(b_hbm.at[idxb_v.at[pl.ds(0, G)]],
                                  brow_v.at[slot], sem_b).wait()

        def wait_store(slot):
            pltpu.make_async_copy(prob_v.at[slot], prob_hbm.at[0],
                                  sem_sp).wait()
            pltpu.make_async_copy(fidx_v.at[slot], fidx_hbm.at[0],
                                  sem_sf).wait()

        prefetch(0, 0)

        def chunk_body(k, carry):
            p = lax.rem(k, 2)
            q = 1 - p

            @pl.when(k + 1 < nchunk)
            def _():
                prefetch(k + 1, q)

            wait_gather(p)

            @pl.when(k >= 2)
            def _():
                wait_store(p)

            def group_body(g, carry2):
                e0 = g * L
                for ee in range(L):
                    acc = None
                    for j in range(h // L):
                        va = arow_v[p, e0 + ee, pl.ds(j * L, L)]
                        vb = brow_v[p, e0 + ee, pl.ds(j * L, L)]
                        x = va + vb
                        el = jnp.where(x > 0, x, jnp.exp(x) - 1.0)
                        t = el * w2_v[pl.ds(j * L, L)]
                        acc = t if acc is None else acc + t
                    m_v[ee] = acc
                s = None
                for j in range(L):
                    v = plsc.load_gather(
                        m_v, [rowi, jnp.full((L,), j, jnp.int32)])
                    s = v if s is None else s + v
                logit = s + b2v
                pv = 1.0 / (1.0 + jnp.exp(-logit))
                ra = idxa_v[pl.ds(k * G + e0, L)]
                rb = idxb_v[pl.ds(k * G + e0, L)]
                pv = jnp.where(ra == rb, jnp.float32(1.0), pv)
                prob_v[p, pl.ds(e0, L)] = pv
                fidx_v[p, pl.ds(e0, L)] = ra * n + rb
                return carry2

            lax.fori_loop(0, G // L, group_body, 0)
            crow = cbase + k
            pltpu.async_copy(prob_v.at[p], prob_hbm.at[crow], sem_sp)
            pltpu.async_copy(fidx_v.at[p], fidx_hbm.at[crow], sem_sf)
            return carry

        lax.fori_loop(0, nchunk, chunk_body, 0)
        wait_store(0)
        wait_store(1)

    prob, fidx = edge_kernel(a_mat, b_mat, row_p, col_p, w2_vec, b2_vec)

    blk_i = 4_194_304
    adj0 = pl.pallas_call(
        _init_body,
        grid=(pl.cdiv(n * n, blk_i),),
        out_specs=pl.BlockSpec((blk_i,), lambda i: (i,)),
        out_shape=jax.ShapeDtypeStruct((n * n,), jnp.float32),
    )()

    diag_fidx = ((jnp.arange(dchunks * G, dtype=jnp.int32) % n)
                 * (n + 1)).reshape(dchunks, G)
    diag_prob = jnp.ones((dchunks, G), jnp.float32)
    sprob = jnp.concatenate([prob, diag_prob], axis=0)
    sfidx = jnp.concatenate([fidx, diag_fidx], axis=0)

    @functools.partial(
        pl.kernel,
        mesh=mesh,
        out_type=(),
        scratch_types=[
            pltpu.VMEM((spw, G), jnp.float32),
            pltpu.VMEM((spw, G), jnp.int32),
            pltpu.SemaphoreType.DMA,
            pltpu.SemaphoreType.DMA,
        ],
    )
    def scatter_kernel(prob_hbm, fidx_hbm, adj_hbm, pv, iv, sem_l, sem_s):
        cid = lax.axis_index("c")
        sid = lax.axis_index("s")
        cbase = jnp.where(cid == 0, sid * SN0, 16 * SN0 + sid * SN1)
        ccount = jnp.where(cid == 0, SN0, SN1)

        def issue_load(k, carry):
            pltpu.async_copy(prob_hbm.at[cbase + k], pv.at[k], sem_l)
            pltpu.async_copy(fidx_hbm.at[cbase + k], iv.at[k], sem_l)
            return carry

        def drain_load(k, carry):
            pltpu.make_async_copy(prob_hbm.at[0], pv.at[0], sem_l).wait()
            pltpu.make_async_copy(fidx_hbm.at[0], iv.at[0], sem_l).wait()
            return carry

        def issue_scatter(k, carry):
            pltpu.async_copy(pv.at[k], adj_hbm.at[iv.at[k]], sem_s)
            return carry

        def drain_scatter(k, carry):
            pltpu.make_async_copy(pv.at[0], adj_hbm.at[iv.at[0]],
                                  sem_s).wait()
            return carry

        lax.fori_loop(0, ccount, issue_load, 0)
        lax.fori_loop(0, ccount, drain_load, 0)
        lax.fori_loop(0, ccount, issue_scatter, 0)
        lax.fori_loop(0, ccount, drain_scatter, 0)

    adj_ref = jax.new_ref(adj0)
    scatter_kernel(sprob, sfidx, adj_ref)
    return jax.freeze(adj_ref).reshape(n, n)
